# baseline (speedup 1.0000x reference)
import jax
import jax.numpy as jnp
from jax.experimental import pallas as pl

AXON = 1000
NEURON = 10000
T_STEPS = 16
BETA = 0.95
THRESH = 1.0
B = 128

NT = 512


def _lif_body(x_ref, w_ref, m_ref, b_ref, spk_ref, mem_ref):
    wm = w_ref[...] * m_ref[...].astype(jnp.float32)
    cur = jax.lax.dot_general(
        x_ref[...], wm,
        dimension_numbers=(((1,), (1,)), ((), ())),
        preferred_element_type=jnp.float32,
    ) + b_ref[...]
    mem = jnp.zeros_like(cur)
    for t in range(T_STEPS):
        reset = (mem > THRESH).astype(jnp.float32)
        mem = BETA * mem + cur - reset * THRESH
        spk_ref[t] = (mem > THRESH).astype(jnp.float32)
        mem_ref[t] = mem


def kernel(x, W, b, mask):
    b2 = b.reshape(1, NEURON)
    grid = (pl.cdiv(NEURON, NT),)
    spk, mem = pl.pallas_call(
        _lif_body,
        grid=grid,
        in_specs=[
            pl.BlockSpec((B, AXON), lambda i: (0, 0)),
            pl.BlockSpec((NT, AXON), lambda i: (i, 0)),
            pl.BlockSpec((NT, AXON), lambda i: (i, 0)),
            pl.BlockSpec((1, NT), lambda i: (0, i)),
        ],
        out_specs=[
            pl.BlockSpec((T_STEPS, B, NT), lambda i: (0, 0, i)),
            pl.BlockSpec((T_STEPS, B, NT), lambda i: (0, 0, i)),
        ],
        out_shape=[
            jax.ShapeDtypeStruct((T_STEPS, B, NEURON), jnp.float32),
            jax.ShapeDtypeStruct((T_STEPS, B, NEURON), jnp.float32),
        ],
    )(x, W, mask, b2)
    return spk, mem


# speedup vs baseline: 1.0069x; 1.0069x over previous
import jax
import jax.numpy as jnp
from jax.experimental import pallas as pl

AXON = 1000
NEURON = 10000
T_STEPS = 16
BETA = 0.95
THRESH = 1.0
B = 128

NT = 1024


def _lif_body(x_ref, w_ref, m_ref, b_ref, spk_ref, mem_ref):
    wm = w_ref[...] * m_ref[...].astype(jnp.float32)
    cur = jax.lax.dot_general(
        x_ref[...], wm,
        dimension_numbers=(((1,), (1,)), ((), ())),
        preferred_element_type=jnp.float32,
    ) + b_ref[...]
    mem = jnp.zeros_like(cur)
    for t in range(T_STEPS):
        reset = (mem > THRESH).astype(jnp.float32)
        mem = BETA * mem + cur - reset * THRESH
        spk_ref[t] = (mem > THRESH).astype(jnp.float32)
        mem_ref[t] = mem


def kernel(x, W, b, mask):
    b2 = b.reshape(1, NEURON)
    grid = (pl.cdiv(NEURON, NT),)
    spk, mem = pl.pallas_call(
        _lif_body,
        grid=grid,
        in_specs=[
            pl.BlockSpec((B, AXON), lambda i: (0, 0)),
            pl.BlockSpec((NT, AXON), lambda i: (i, 0)),
            pl.BlockSpec((NT, AXON), lambda i: (i, 0)),
            pl.BlockSpec((1, NT), lambda i: (0, i)),
        ],
        out_specs=[
            pl.BlockSpec((T_STEPS, B, NT), lambda i: (0, 0, i)),
            pl.BlockSpec((T_STEPS, B, NT), lambda i: (0, 0, i)),
        ],
        out_shape=[
            jax.ShapeDtypeStruct((T_STEPS, B, NEURON), jnp.float32),
            jax.ShapeDtypeStruct((T_STEPS, B, NEURON), jnp.float32),
        ],
    )(x, W, mask, b2)
    return spk, mem
